# baseline (speedup 1.0000x reference)
import functools

import jax
import jax.numpy as jnp
import numpy as np
from jax import lax
from jax.experimental import pallas as pl
from jax.experimental.pallas import tpu as pltpu
from jax.experimental.pallas import tpu_sc as plsc

ROWS, COLS = 4096, 8192
R_TC = 3584
R_SC = ROWS - R_TC

L = 16
NC, NS = 2, 16
NW = NC * NS
RPW = R_SC // NW
RB = 8
NBLK = RPW // RB
VPR = COLS // L

_GDN = lax.GatherDimensionNumbers(
    offset_dims=(), collapsed_slice_dims=(0,), start_index_map=(0,))


def _bcast_last(s):
    last = jnp.full((L, 1), L - 1, jnp.int32)
    return lax.gather(s, last, _GDN, slice_sizes=(1,),
                      mode=lax.GatherScatterMode.PROMISE_IN_BOUNDS)


def _rc_rows2(buf, r0, r1):
    def step(k, us):
        u0, u1 = us
        base = (VPR - 1 - k) * L
        v0 = buf[r0, pl.ds(base, L)]
        v1 = buf[r1, pl.ds(base, L)]
        s0 = plsc.cumsum(v0)
        s1 = plsc.cumsum(v1)
        t0 = u0 + _bcast_last(s0)
        t1 = u1 + _bcast_last(s1)
        buf[r0, pl.ds(base, L)] = t0 - s0 + v0
        buf[r1, pl.ds(base, L)] = t1 - s1 + v1
        return (t0, t1)

    z = jnp.zeros((L,), jnp.float32)
    lax.fori_loop(0, VPR, step, (z, z), unroll=4)


def _sc_body(x_hbm, out_hbm, buf):
    wid = lax.axis_index("s") * NC + lax.axis_index("c")
    row_base = wid * RPW

    def block(b, carry):
        r0 = row_base + b * RB
        pltpu.sync_copy(x_hbm.at[pl.ds(R_TC + r0, RB)], buf)
        for r in range(0, RB, 2):
            _rc_rows2(buf, r, r + 1)
        pltpu.sync_copy(buf, out_hbm.at[pl.ds(r0, RB)])
        return carry

    lax.fori_loop(0, NBLK, block, 0)


def _sc_part(x):
    mesh = plsc.VectorSubcoreMesh(core_axis_name="c", subcore_axis_name="s")
    f = pl.kernel(
        _sc_body,
        out_type=jax.ShapeDtypeStruct((R_SC, COLS), jnp.float32),
        mesh=mesh,
        scratch_types=[pltpu.VMEM((RB, COLS), jnp.float32)],
        compiler_params=pltpu.CompilerParams(
            needs_layout_passes=False, use_tc_tiling_on_sc=True),
    )
    return f(x)


BR = 3584
W = 512
C = COLS // W


def _tc_kernel(tri_ref, x_ref, o_ref, carry_ref):
    j = pl.program_id(1)

    @pl.when(j == 0)
    def _():
        carry_ref[...] = jnp.zeros_like(carry_ref)

    part = jax.lax.dot_general(
        x_ref[...].astype(jnp.bfloat16), tri_ref[...].astype(jnp.bfloat16),
        (((1,), (0,)), ((), ())), preferred_element_type=jnp.float32)
    o_ref[...] = part + carry_ref[:, 0:1]
    carry_ref[...] = jnp.broadcast_to(
        part[:, 0:1] + carry_ref[:, 0:1], carry_ref.shape)


def _tc_part(tri, x):
    return pl.pallas_call(
        _tc_kernel,
        grid=(R_TC // BR, C),
        in_specs=[
            pl.BlockSpec((W, W), lambda i, j: (0, 0)),
            pl.BlockSpec((BR, W), lambda i, j: (i, C - 1 - j)),
        ],
        out_specs=pl.BlockSpec((BR, W), lambda i, j: (i, C - 1 - j)),
        out_shape=jax.ShapeDtypeStruct((ROWS, COLS), jnp.float32),
        scratch_shapes=[pltpu.VMEM((BR, 128), jnp.float32)],
        compiler_params=pltpu.CompilerParams(
            dimension_semantics=("parallel", "arbitrary")),
    )(tri, x)


BRS = 128


def _splice_kernel(tc_ref, sc_ref, o_ref):
    o_ref[...] = sc_ref[...]


def _splice(tc_out, sc_part):
    return pl.pallas_call(
        _splice_kernel,
        grid=(R_SC // BRS,),
        in_specs=[
            pl.BlockSpec(memory_space=pl.ANY),
            pl.BlockSpec((BRS, COLS), lambda i: (i, 0)),
        ],
        out_specs=pl.BlockSpec((BRS, COLS), lambda i: (R_TC // BRS + i, 0)),
        out_shape=jax.ShapeDtypeStruct((ROWS, COLS), jnp.float32),
        input_output_aliases={0: 0},
    )(tc_out, sc_part)


def kernel(x):
    tri = jnp.asarray(np.tril(np.ones((W, W), np.float32)))
    sc_out = _sc_part(x)
    tc_out = _tc_part(tri, x)
    return _splice(tc_out, sc_out)


# speedup vs baseline: 1.0196x; 1.0196x over previous
import functools

import jax
import jax.numpy as jnp
import numpy as np
from jax import lax
from jax.experimental import pallas as pl
from jax.experimental.pallas import tpu as pltpu
from jax.experimental.pallas import tpu_sc as plsc

ROWS, COLS = 4096, 8192
R_TC = 3584
R_SC = ROWS - R_TC

L = 16
NC, NS = 2, 16
NW = NC * NS
RPW = R_SC // NW
RB = 8
NBLK = RPW // RB
VPR = COLS // L

_GDN = lax.GatherDimensionNumbers(
    offset_dims=(), collapsed_slice_dims=(0,), start_index_map=(0,))


def _bcast_last(s):
    last = jnp.full((L, 1), L - 1, jnp.int32)
    return lax.gather(s, last, _GDN, slice_sizes=(1,),
                      mode=lax.GatherScatterMode.PROMISE_IN_BOUNDS)


def _rc_rows2(buf, r0, r1):
    def step(k, us):
        u0, u1 = us
        base = (VPR - 1 - k) * L
        v0 = buf[r0, pl.ds(base, L)]
        v1 = buf[r1, pl.ds(base, L)]
        s0 = plsc.cumsum(v0)
        s1 = plsc.cumsum(v1)
        t0 = u0 + _bcast_last(s0)
        t1 = u1 + _bcast_last(s1)
        buf[r0, pl.ds(base, L)] = t0 - s0 + v0
        buf[r1, pl.ds(base, L)] = t1 - s1 + v1
        return (t0, t1)

    z = jnp.zeros((L,), jnp.float32)
    lax.fori_loop(0, VPR, step, (z, z), unroll=4)


def _sc_body(x_hbm, out_hbm, buf):
    wid = lax.axis_index("s") * NC + lax.axis_index("c")
    row_base = wid * RPW

    def block(b, carry):
        r0 = row_base + b * RB
        pltpu.sync_copy(x_hbm.at[pl.ds(R_TC + r0, RB)], buf)
        for r in range(0, RB, 2):
            _rc_rows2(buf, r, r + 1)
        pltpu.sync_copy(buf, out_hbm.at[pl.ds(r0, RB)])
        return carry

    lax.fori_loop(0, NBLK, block, 0)


def _sc_part(x):
    mesh = plsc.VectorSubcoreMesh(core_axis_name="c", subcore_axis_name="s")
    f = pl.kernel(
        _sc_body,
        out_type=jax.ShapeDtypeStruct((R_SC, COLS), jnp.float32),
        mesh=mesh,
        scratch_types=[pltpu.VMEM((RB, COLS), jnp.float32)],
        compiler_params=pltpu.CompilerParams(
            needs_layout_passes=False, use_tc_tiling_on_sc=True),
    )
    return f(x)


BR = 1792
W = 512
C = COLS // W


def _tc_kernel(tri_ref, x_ref, o_ref, carry_ref):
    j = pl.program_id(1)

    @pl.when(j == 0)
    def _():
        carry_ref[...] = jnp.zeros_like(carry_ref)

    part = jax.lax.dot_general(
        x_ref[...].astype(jnp.bfloat16), tri_ref[...].astype(jnp.bfloat16),
        (((1,), (0,)), ((), ())), preferred_element_type=jnp.float32)
    o_ref[...] = part + carry_ref[:, 0:1]
    carry_ref[...] = jnp.broadcast_to(
        part[:, 0:1] + carry_ref[:, 0:1], carry_ref.shape)


def _tc_part(tri, x):
    return pl.pallas_call(
        _tc_kernel,
        grid=(R_TC // BR, C),
        in_specs=[
            pl.BlockSpec((W, W), lambda i, j: (0, 0)),
            pl.BlockSpec((BR, W), lambda i, j: (i, C - 1 - j)),
        ],
        out_specs=pl.BlockSpec((BR, W), lambda i, j: (i, C - 1 - j)),
        out_shape=jax.ShapeDtypeStruct((ROWS, COLS), jnp.float32),
        scratch_shapes=[pltpu.VMEM((BR, 128), jnp.float32)],
        compiler_params=pltpu.CompilerParams(
            dimension_semantics=("parallel", "arbitrary")),
    )(tri, x)


BRS = 128


def _splice_kernel(tc_ref, sc_ref, o_ref):
    o_ref[...] = sc_ref[...]


def _splice(tc_out, sc_part):
    return pl.pallas_call(
        _splice_kernel,
        grid=(R_SC // BRS,),
        in_specs=[
            pl.BlockSpec(memory_space=pl.ANY),
            pl.BlockSpec((BRS, COLS), lambda i: (i, 0)),
        ],
        out_specs=pl.BlockSpec((BRS, COLS), lambda i: (R_TC // BRS + i, 0)),
        out_shape=jax.ShapeDtypeStruct((ROWS, COLS), jnp.float32),
        input_output_aliases={0: 0},
    )(tc_out, sc_part)


def kernel(x):
    tri = jnp.asarray(np.tril(np.ones((W, W), np.float32)))
    sc_out = _sc_part(x)
    tc_out = _tc_part(tri, x)
    return _splice(tc_out, sc_out)
